# initial kernel scaffold (unmeasured)
import jax
import jax.numpy as jnp
from jax import lax
from jax.experimental import pallas as pl
from jax.experimental.pallas import tpu as pltpu


def kernel(
    x,
):
    def body(*refs):
        pass

    out_shape = jax.ShapeDtypeStruct(..., jnp.float32)
    return pl.pallas_call(body, out_shape=out_shape)(...)


# baseline (device time: 13484 ns/iter reference)
import jax
import jax.numpy as jnp
from jax import lax
from jax.experimental import pallas as pl
from jax.experimental.pallas import tpu as pltpu


def kernel(x):
    m, n = x.shape

    def body(x_ref, out_ref, comm_ref, sum_ref, send_sems, recv_sems):
        my_x = lax.axis_index("x")
        my_y = lax.axis_index("y")

        barrier_sem = pltpu.get_barrier_semaphore()
        pl.semaphore_signal(
            barrier_sem, inc=1,
            device_id=(1 - my_x, my_y), device_id_type=pl.DeviceIdType.MESH,
        )
        pl.semaphore_signal(
            barrier_sem, inc=1,
            device_id=(my_x, 1 - my_y), device_id_type=pl.DeviceIdType.MESH,
        )
        pl.semaphore_wait(barrier_sem, 2)

        rdma_x = pltpu.make_async_remote_copy(
            src_ref=x_ref,
            dst_ref=comm_ref.at[0],
            send_sem=send_sems.at[0],
            recv_sem=recv_sems.at[0],
            device_id=(1 - my_x, my_y),
            device_id_type=pl.DeviceIdType.MESH,
        )
        rdma_x.start()
        rdma_x.wait()

        sum_ref[:, :] = x_ref[:, :] + comm_ref[0]

        rdma_y = pltpu.make_async_remote_copy(
            src_ref=sum_ref,
            dst_ref=comm_ref.at[1],
            send_sem=send_sems.at[1],
            recv_sem=recv_sems.at[1],
            device_id=(my_x, 1 - my_y),
            device_id_type=pl.DeviceIdType.MESH,
        )
        rdma_y.start()
        rdma_y.wait()

        out_ref[:, pl.ds(my_y * n, n)] = sum_ref[:, :]
        out_ref[:, pl.ds((1 - my_y) * n, n)] = comm_ref[1]

    return pl.pallas_call(
        body,
        out_shape=jax.ShapeDtypeStruct((m, 2 * n), x.dtype),
        in_specs=[pl.BlockSpec(memory_space=pltpu.VMEM)],
        out_specs=pl.BlockSpec(memory_space=pltpu.VMEM),
        scratch_shapes=[
            pltpu.VMEM((2, m, n), x.dtype),
            pltpu.VMEM((m, n), x.dtype),
            pltpu.SemaphoreType.DMA((2,)),
            pltpu.SemaphoreType.DMA((2,)),
        ],
        compiler_params=pltpu.CompilerParams(collective_id=0),
    )(x)


# device time: 11468 ns/iter; 1.1758x vs baseline; 1.1758x over previous
import jax
import jax.numpy as jnp
from jax import lax
from jax.experimental import pallas as pl
from jax.experimental.pallas import tpu as pltpu

C = 4


def kernel(x):
    m, n = x.shape
    mc = m // C

    def body(x_ref, out_ref, commx, commy, sum_ref,
             sendx, recvx, sendy, recvy):
        my_x = lax.axis_index("x")
        my_y = lax.axis_index("y")
        xp = (1 - my_x, my_y)
        yp = (my_x, 1 - my_y)

        barrier_sem = pltpu.get_barrier_semaphore()
        pl.semaphore_signal(
            barrier_sem, inc=1, device_id=xp,
            device_id_type=pl.DeviceIdType.MESH,
        )
        pl.semaphore_signal(
            barrier_sem, inc=1, device_id=yp,
            device_id_type=pl.DeviceIdType.MESH,
        )
        pl.semaphore_wait(barrier_sem, 2)

        xr = []
        for c in range(C):
            r = pltpu.make_async_remote_copy(
                src_ref=x_ref.at[pl.ds(c * mc, mc)],
                dst_ref=commx.at[c],
                send_sem=sendx.at[c],
                recv_sem=recvx.at[c],
                device_id=xp,
                device_id_type=pl.DeviceIdType.MESH,
            )
            r.start()
            xr.append(r)

        yr = []
        for c in range(C):
            xr[c].wait_recv()
            sum_ref[c] = x_ref[pl.ds(c * mc, mc), :] + commx[c]
            r = pltpu.make_async_remote_copy(
                src_ref=sum_ref.at[c],
                dst_ref=commy.at[c],
                send_sem=sendy.at[c],
                recv_sem=recvy.at[c],
                device_id=yp,
                device_id_type=pl.DeviceIdType.MESH,
            )
            r.start()
            yr.append(r)
            out_ref[pl.ds(c * mc, mc), pl.ds(my_y * n, n)] = sum_ref[c]

        for c in range(C):
            yr[c].wait_recv()
            out_ref[pl.ds(c * mc, mc), pl.ds((1 - my_y) * n, n)] = commy[c]

        for c in range(C):
            xr[c].wait_send()
            yr[c].wait_send()

    return pl.pallas_call(
        body,
        out_shape=jax.ShapeDtypeStruct((m, 2 * n), x.dtype),
        in_specs=[pl.BlockSpec(memory_space=pltpu.VMEM)],
        out_specs=pl.BlockSpec(memory_space=pltpu.VMEM),
        scratch_shapes=[
            pltpu.VMEM((C, mc, n), x.dtype),
            pltpu.VMEM((C, mc, n), x.dtype),
            pltpu.VMEM((C, mc, n), x.dtype),
            pltpu.SemaphoreType.DMA((C,)),
            pltpu.SemaphoreType.DMA((C,)),
            pltpu.SemaphoreType.DMA((C,)),
            pltpu.SemaphoreType.DMA((C,)),
        ],
        compiler_params=pltpu.CompilerParams(collective_id=0),
    )(x)


# device time: 11428 ns/iter; 1.1799x vs baseline; 1.0035x over previous
import jax
import jax.numpy as jnp
from jax import lax
from jax.experimental import pallas as pl
from jax.experimental.pallas import tpu as pltpu

C = 4


def kernel(x):
    m, n = x.shape
    mc = m // C

    def body(x_ref, out_ref, commx, sendx, recvx, sendy, recvy):
        my_x = lax.axis_index("x")
        my_y = lax.axis_index("y")
        xp = (1 - my_x, my_y)
        yp = (my_x, 1 - my_y)

        barrier_sem = pltpu.get_barrier_semaphore()
        pl.semaphore_signal(
            barrier_sem, inc=1, device_id=xp,
            device_id_type=pl.DeviceIdType.MESH,
        )
        pl.semaphore_signal(
            barrier_sem, inc=1, device_id=yp,
            device_id_type=pl.DeviceIdType.MESH,
        )
        pl.semaphore_wait(barrier_sem, 2)

        xr = []
        for c in range(C):
            r = pltpu.make_async_remote_copy(
                src_ref=x_ref.at[pl.ds(c * mc, mc)],
                dst_ref=commx.at[c],
                send_sem=sendx.at[c],
                recv_sem=recvx.at[c],
                device_id=xp,
                device_id_type=pl.DeviceIdType.MESH,
            )
            r.start()
            xr.append(r)

        yr = []
        for c in range(C):
            xr[c].wait_recv()
            out_ref[pl.ds(c * mc, mc), pl.ds(my_y * n, n)] = (
                x_ref[pl.ds(c * mc, mc), :] + commx[c]
            )
            r = pltpu.make_async_remote_copy(
                src_ref=out_ref.at[pl.ds(c * mc, mc), pl.ds(my_y * n, n)],
                dst_ref=out_ref.at[pl.ds(c * mc, mc), pl.ds(my_y * n, n)],
                send_sem=sendy.at[c],
                recv_sem=recvy.at[c],
                device_id=yp,
                device_id_type=pl.DeviceIdType.MESH,
            )
            r.start()
            yr.append(r)

        for c in range(C):
            yr[c].wait_recv()

        for c in range(C):
            xr[c].wait_send()
            yr[c].wait_send()

    return pl.pallas_call(
        body,
        out_shape=jax.ShapeDtypeStruct((m, 2 * n), x.dtype),
        in_specs=[pl.BlockSpec(memory_space=pltpu.VMEM)],
        out_specs=pl.BlockSpec(memory_space=pltpu.VMEM),
        scratch_shapes=[
            pltpu.VMEM((C, mc, n), x.dtype),
            pltpu.SemaphoreType.DMA((C,)),
            pltpu.SemaphoreType.DMA((C,)),
            pltpu.SemaphoreType.DMA((C,)),
            pltpu.SemaphoreType.DMA((C,)),
        ],
        compiler_params=pltpu.CompilerParams(collective_id=0),
    )(x)


# device time: 11256 ns/iter; 1.1979x vs baseline; 1.0153x over previous
import jax
import jax.numpy as jnp
from jax import lax
from jax.experimental import pallas as pl
from jax.experimental.pallas import tpu as pltpu

CHUNKS = [32] * 8
OFFSETS = [32 * i for i in range(8)]


def kernel(x):
    m, n = x.shape
    assert sum(CHUNKS) == m
    C = len(CHUNKS)

    def body(x_ref, out_ref, commx, sendx, recvx, sendy, recvy):
        my_x = lax.axis_index("x")
        my_y = lax.axis_index("y")
        xp = (1 - my_x, my_y)
        yp = (my_x, 1 - my_y)

        barrier_sem = pltpu.get_barrier_semaphore()
        pl.semaphore_signal(
            barrier_sem, inc=1, device_id=xp,
            device_id_type=pl.DeviceIdType.MESH,
        )
        pl.semaphore_signal(
            barrier_sem, inc=1, device_id=yp,
            device_id_type=pl.DeviceIdType.MESH,
        )
        pl.semaphore_wait(barrier_sem, 2)

        xr = []
        for c, (off, sz) in enumerate(zip(OFFSETS, CHUNKS)):
            r = pltpu.make_async_remote_copy(
                src_ref=x_ref.at[pl.ds(off, sz)],
                dst_ref=commx.at[pl.ds(off, sz)],
                send_sem=sendx.at[c],
                recv_sem=recvx.at[c],
                device_id=xp,
                device_id_type=pl.DeviceIdType.MESH,
            )
            r.start()
            xr.append(r)

        yr = []
        for c, (off, sz) in enumerate(zip(OFFSETS, CHUNKS)):
            xr[c].wait_recv()
            out_ref[pl.ds(off, sz), pl.ds(my_y * n, n)] = (
                x_ref[pl.ds(off, sz), :] + commx[pl.ds(off, sz), :]
            )
            r = pltpu.make_async_remote_copy(
                src_ref=out_ref.at[pl.ds(off, sz), pl.ds(my_y * n, n)],
                dst_ref=out_ref.at[pl.ds(off, sz), pl.ds(my_y * n, n)],
                send_sem=sendy.at[c],
                recv_sem=recvy.at[c],
                device_id=yp,
                device_id_type=pl.DeviceIdType.MESH,
            )
            r.start()
            yr.append(r)

        for c in range(C):
            yr[c].wait_recv()

        for c in range(C):
            xr[c].wait_send()
            yr[c].wait_send()

    C = len(CHUNKS)
    return pl.pallas_call(
        body,
        out_shape=jax.ShapeDtypeStruct((m, 2 * n), x.dtype),
        in_specs=[pl.BlockSpec(memory_space=pltpu.VMEM)],
        out_specs=pl.BlockSpec(memory_space=pltpu.VMEM),
        scratch_shapes=[
            pltpu.VMEM((m, n), x.dtype),
            pltpu.SemaphoreType.DMA((C,)),
            pltpu.SemaphoreType.DMA((C,)),
            pltpu.SemaphoreType.DMA((C,)),
            pltpu.SemaphoreType.DMA((C,)),
        ],
        compiler_params=pltpu.CompilerParams(collective_id=0),
    )(x)
